# baseline (speedup 1.0000x reference)
import jax
import jax.numpy as jnp
from jax import lax
from jax.experimental import pallas as pl
from jax.experimental.pallas import tpu as pltpu
from jax.experimental.pallas import tpu_sc as plsc

B = 16384
K_NUM = 13
K_CAT = 26
T = K_NUM + K_CAT
D = 32
CARD = 100000
EROWS = K_CAT * CARD
LPR = 128 // D

NC = 2
NS = 16
NW = NC * NS
RW = B // NW
CB = 16
NCHUNK = RW // CB
NPAIR = NCHUNK // 2
NCAT = CB * K_CAT
NROW4 = CB * T // 4

_LANES = 16
_UNROLL = 4

GCH = 104
NG = NCAT // GCH


def _tokenizer_body(x_hbm, e_hbm, bc_hbm, w_hbm, bvec_hbm, out_hbm,
                    xv, line_v, colb_v, cat_lines, obuf0, obuf1,
                    wv, bv, bcv, gsem, osem0, osem1):
  wid = lax.axis_index("c") * NS + lax.axis_index("s")
  wbase = wid * RW

  pltpu.sync_copy(w_hbm, wv)
  pltpu.sync_copy(bvec_hbm, bv)
  pltpu.sync_copy(bc_hbm, bcv)

  lanes = lax.iota(jnp.int32, _LANES)
  offs_lo = lanes * CARD
  offs_hi = (10 + lanes) * CARD

  obufs = (obuf0, obuf1)
  osems = (osem0, osem1)

  def do_chunk(c, first, obuf, osem):
    base = wbase + c * CB
    pltpu.sync_copy(x_hbm.at[pl.ds(base, CB), :], xv)

    def idx_row(i, _):
      a = xv[i, pl.ds(13, _LANES)].astype(jnp.int32) + offs_lo
      h = xv[i, pl.ds(23, _LANES)].astype(jnp.int32) + offs_hi
      line_v[pl.ds(i * K_CAT, _LANES)] = lax.shift_right_logical(a, 2)
      line_v[pl.ds(i * K_CAT + 10, _LANES)] = lax.shift_right_logical(h, 2)
      colb_v[pl.ds(i * K_CAT, _LANES)] = lax.shift_left(a & 3, 5)
      colb_v[pl.ds(i * K_CAT + 10, _LANES)] = lax.shift_left(h & 3, 5)
      return 0

    lax.fori_loop(0, CB, idx_row, 0, unroll=_UNROLL)

    copies = []
    for g in range(NG):
      copies.append(
          pltpu.async_copy(
              e_hbm.at[line_v.at[pl.ds(g * GCH, GCH)]],
              cat_lines.at[pl.ds(g * GCH, GCH), :],
              gsem,
          ))

    @pl.when(jnp.logical_not(first))
    def _():
      pltpu.make_async_copy(out_hbm.at[pl.ds(0, NROW4), :], obuf, osem).wait()

    for k in range(K_NUM):
      wlo = wv[k, pl.ds(0, _LANES)]
      whi = wv[k, pl.ds(16, _LANES)]
      blo = bv[k, pl.ds(0, _LANES)]
      bhi = bv[k, pl.ds(16, _LANES)]

      def num_row(i, _, k=k, wlo=wlo, whi=whi, blo=blo, bhi=bhi):
        xn = xv[i, pl.ds(0, _LANES)]
        sv = jnp.full((_LANES,), xn[k], jnp.float32)
        t = i * T + k
        obuf[t // 4, pl.ds((t % 4) * D, _LANES)] = (
            jnp.maximum(sv * wlo + blo, 0.0))
        obuf[t // 4, pl.ds((t % 4) * D + 16, _LANES)] = (
            jnp.maximum(sv * whi + bhi, 0.0))
        return 0

      lax.fori_loop(0, CB, num_row, 0)

    for cp in copies:
      cp.wait()

    def cat_group(pg, _):
      cb16 = colb_v[pl.ds(pg * _LANES, _LANES)]
      for lane in range(_LANES):
        p = pg * _LANES + lane
        i = p // K_CAT
        j = p - i * K_CAT
        colb = cb16[lane]
        vlo = cat_lines[p, pl.ds(colb, _LANES)]
        vhi = cat_lines[p, pl.ds(colb + 16, _LANES)]
        t = i * T + K_NUM + j
        obuf[t // 4, pl.ds((t % 4) * D, _LANES)] = vlo + bcv[j, pl.ds(0, _LANES)]
        obuf[t // 4, pl.ds((t % 4) * D + 16, _LANES)] = (
            vhi + bcv[j, pl.ds(16, _LANES)])
      return 0

    lax.fori_loop(0, NCAT // _LANES, cat_group, 0)

    pltpu.async_copy(obuf, out_hbm.at[pl.ds(base * T // 4, NROW4), :], osem)

  def pair_body(p, _):
    do_chunk(2 * p, p == 0, obufs[0], osems[0])
    do_chunk(2 * p + 1, p == 0, obufs[1], osems[1])
    return 0

  lax.fori_loop(0, NPAIR, pair_body, 0)

  for h in range(2):
    pltpu.make_async_copy(out_hbm.at[pl.ds(0, NROW4), :], obufs[h],
                          osems[h]).wait()


@jax.jit
def kernel(x, E, bc, W, b, lookup_idx):
  del lookup_idx
  e_lines = E.reshape(EROWS // LPR, 128)
  mesh = plsc.VectorSubcoreMesh(core_axis_name="c", subcore_axis_name="s")
  out = pl.kernel(
      _tokenizer_body,
      out_type=jax.ShapeDtypeStruct((B * T // 4, 128), jnp.float32),
      mesh=mesh,
      compiler_params=pltpu.CompilerParams(use_tc_tiling_on_sc=False),
      scratch_types=[
          pltpu.VMEM((CB, T), jnp.float32),
          pltpu.VMEM((NCAT,), jnp.int32),
          pltpu.VMEM((NCAT,), jnp.int32),
          pltpu.VMEM((NCAT, 128), jnp.float32),
          pltpu.VMEM((NROW4, 128), jnp.float32),
          pltpu.VMEM((NROW4, 128), jnp.float32),
          pltpu.VMEM((K_NUM, D), jnp.float32),
          pltpu.VMEM((K_NUM, D), jnp.float32),
          pltpu.VMEM((K_CAT, D), jnp.float32),
          pltpu.SemaphoreType.DMA,
          pltpu.SemaphoreType.DMA,
          pltpu.SemaphoreType.DMA,
      ],
  )(x, e_lines, bc, W, b)
  return out.reshape(B, T, D)


# speedup vs baseline: 1.0421x; 1.0421x over previous
import jax
import jax.numpy as jnp
from jax import lax
from jax.experimental import pallas as pl
from jax.experimental.pallas import tpu as pltpu
from jax.experimental.pallas import tpu_sc as plsc

B = 16384
K_NUM = 13
K_CAT = 26
T = K_NUM + K_CAT
D = 32
CARD = 100000
EROWS = K_CAT * CARD
LPR = 128 // D

NC = 2
NS = 16
NW = NC * NS
RW = B // NW
CB = 32
NCHUNK = RW // CB
NPAIR = NCHUNK // 2
NCAT = CB * K_CAT
NROW4 = CB * T // 4

SPB = 8
NSP = CB // SPB
SPCAT = SPB * K_CAT

_LANES = 16
_UNROLL = 4

GCH = 104
NG = SPCAT // GCH


def _tokenizer_body(x_hbm, e_hbm, bc_hbm, w_hbm, bvec_hbm, out_hbm,
                    xv, line_v, colb_v, cat_lines, obuf0, obuf1,
                    wv, bv, bcv, gsem, osem0, osem1):
  wid = lax.axis_index("c") * NS + lax.axis_index("s")
  wbase = wid * RW

  pltpu.sync_copy(w_hbm, wv)
  pltpu.sync_copy(bvec_hbm, bv)
  pltpu.sync_copy(bc_hbm, bcv)

  lanes = lax.iota(jnp.int32, _LANES)
  offs_lo = lanes * CARD
  offs_hi = (10 + lanes) * CARD

  obufs = (obuf0, obuf1)
  osems = (osem0, osem1)

  def do_chunk(c, first, obuf, osem):
    base = wbase + c * CB
    pltpu.sync_copy(x_hbm.at[pl.ds(base, CB), :], xv)

    def idx_row(i, _):
      a = xv[i, pl.ds(13, _LANES)].astype(jnp.int32) + offs_lo
      h = xv[i, pl.ds(23, _LANES)].astype(jnp.int32) + offs_hi
      line_v[pl.ds(i * K_CAT, _LANES)] = lax.shift_right_logical(a, 2)
      line_v[pl.ds(i * K_CAT + 10, _LANES)] = lax.shift_right_logical(h, 2)
      colb_v[pl.ds(i * K_CAT, _LANES)] = lax.shift_left(a & 3, 5)
      colb_v[pl.ds(i * K_CAT + 10, _LANES)] = lax.shift_left(h & 3, 5)
      return 0

    lax.fori_loop(0, CB, idx_row, 0, unroll=_UNROLL)

    @pl.when(jnp.logical_not(first))
    def _():
      pltpu.make_async_copy(out_hbm.at[pl.ds(0, NROW4), :], obuf, osem).wait()

    for sp in range(NSP):
      copies = []
      for g in range(NG):
        copies.append(
            pltpu.async_copy(
                e_hbm.at[line_v.at[pl.ds(sp * SPCAT + g * GCH, GCH)]],
                cat_lines.at[pl.ds(g * GCH, GCH), :],
                gsem,
            ))

      if sp == 0:
        for k in range(K_NUM):
          wlo = wv[k, pl.ds(0, _LANES)]
          whi = wv[k, pl.ds(16, _LANES)]
          blo = bv[k, pl.ds(0, _LANES)]
          bhi = bv[k, pl.ds(16, _LANES)]

          def num_row(i, _, k=k, wlo=wlo, whi=whi, blo=blo, bhi=bhi):
            xn = xv[i, pl.ds(0, _LANES)]
            sv = jnp.full((_LANES,), xn[k], jnp.float32)
            t = i * T + k
            obuf[t // 4, pl.ds((t % 4) * D, _LANES)] = (
                jnp.maximum(sv * wlo + blo, 0.0))
            obuf[t // 4, pl.ds((t % 4) * D + 16, _LANES)] = (
                jnp.maximum(sv * whi + bhi, 0.0))
            return 0

          lax.fori_loop(0, CB, num_row, 0)

      for cp in copies:
        cp.wait()

      def cat_group(pg, _, sp=sp):
        cb16 = colb_v[pl.ds(sp * SPCAT + pg * _LANES, _LANES)]
        for lane in range(_LANES):
          src = pg * _LANES + lane
          p = sp * SPCAT + src
          i = p // K_CAT
          j = p - i * K_CAT
          colb = cb16[lane]
          vlo = cat_lines[src, pl.ds(colb, _LANES)]
          vhi = cat_lines[src, pl.ds(colb + 16, _LANES)]
          t = i * T + K_NUM + j
          obuf[t // 4, pl.ds((t % 4) * D, _LANES)] = (
              vlo + bcv[j, pl.ds(0, _LANES)])
          obuf[t // 4, pl.ds((t % 4) * D + 16, _LANES)] = (
              vhi + bcv[j, pl.ds(16, _LANES)])
        return 0

      lax.fori_loop(0, SPCAT // _LANES, cat_group, 0)

    row0 = pl.multiple_of(base * T // 4, 8)
    pltpu.async_copy(obuf, out_hbm.at[pl.ds(row0, NROW4), :], osem)

  def pair_body(p, _):
    do_chunk(2 * p, p == 0, obufs[0], osems[0])
    do_chunk(2 * p + 1, p == 0, obufs[1], osems[1])
    return 0

  lax.fori_loop(0, NPAIR, pair_body, 0)

  for h in range(2):
    pltpu.make_async_copy(out_hbm.at[pl.ds(0, NROW4), :], obufs[h],
                          osems[h]).wait()


@jax.jit
def kernel(x, E, bc, W, b, lookup_idx):
  del lookup_idx
  e_lines = (E.T.reshape(D, EROWS // LPR, LPR)
             .transpose(1, 2, 0).reshape(EROWS // LPR, 128))
  mesh = plsc.VectorSubcoreMesh(core_axis_name="c", subcore_axis_name="s")
  out = pl.kernel(
      _tokenizer_body,
      out_type=jax.ShapeDtypeStruct((B * T // 4, 128), jnp.float32),
      mesh=mesh,
      compiler_params=pltpu.CompilerParams(use_tc_tiling_on_sc=True),
      scratch_types=[
          pltpu.VMEM((CB, T), jnp.float32),
          pltpu.VMEM((NCAT,), jnp.int32),
          pltpu.VMEM((NCAT,), jnp.int32),
          pltpu.VMEM((SPCAT, 128), jnp.float32),
          pltpu.VMEM((NROW4, 128), jnp.float32),
          pltpu.VMEM((NROW4, 128), jnp.float32),
          pltpu.VMEM((K_NUM, D), jnp.float32),
          pltpu.VMEM((K_NUM, D), jnp.float32),
          pltpu.VMEM((K_CAT, D), jnp.float32),
          pltpu.SemaphoreType.DMA,
          pltpu.SemaphoreType.DMA,
          pltpu.SemaphoreType.DMA,
      ],
  )(x, e_lines, bc, W, b)
  return out.reshape(B, T, D)
